# baseline (speedup 1.0000x reference)
import dataclasses
import functools

import jax
import jax.numpy as jnp
from jax import lax
from jax.experimental import pallas as pl
from jax.experimental.pallas import tpu as pltpu
from jax.experimental.pallas import tpu_sc as plsc

N = 10000
E_H = 10000
NNZ = 320000
D_IN = 128
D_H = 128
D_OUT = 64
_EPS_BN = 1e-5

_NW = 32
_EPW = NNZ // _NW
_C = 200
_NCHUNK = _EPW // _C
_NPAD = 10240
_RPS = _NPAD // 16


def _lrelu(x, slope=0.01):
    return jnp.where(x >= 0, x, slope * x)


def _tc_pre_body(x_ref, xe_in_ref, W1_ref, b1_ref, g1_ref, be1_ref,
                 Wr1_ref, br1_ref, Wr2_ref, br2_ref, Wh_ref,
                 atta_ref, attb_ref, Ws_ref, bs_ref, Wc1b_ref, bc1_ref,
                 xt_ref, rp_ref, xep_ref, sn_ref, se_ref):
    f32 = jnp.float32
    xb = _lrelu(jnp.dot(x_ref[...], W1_ref[...], preferred_element_type=f32)
                + b1_ref[...])
    m = jnp.mean(xb, axis=0, keepdims=True)
    v = jnp.mean((xb - m) ** 2, axis=0, keepdims=True)
    xb = (xb - m) / jnp.sqrt(v + _EPS_BN) * g1_ref[...] + be1_ref[...]

    xe1 = _lrelu(jnp.dot(xe_in_ref[...], Wr1_ref[...], preferred_element_type=f32)
                 + br1_ref[...])
    xe = jnp.dot(xe1, Wr2_ref[...], preferred_element_type=f32) + br2_ref[...]

    xt = jnp.dot(xb, Wh_ref[...], preferred_element_type=f32)
    he = jnp.dot(xe, Wh_ref[...], preferred_element_type=f32)

    xt_ref[...] = xt
    sn_ref[...] = jnp.dot(xt, atta_ref[...], preferred_element_type=f32)
    se_ref[...] = jnp.dot(he, attb_ref[...], preferred_element_type=f32)
    rp_ref[...] = jnp.dot(xb, Ws_ref[...], preferred_element_type=f32) + bs_ref[...]
    xep_ref[...] = (jnp.dot(xe, Wc1b_ref[...], preferred_element_type=f32)
                    + bc1_ref[...])


def _tc_pre(x, x_e, W1, b1, g1, be1, Wr1, br1, Wr2, br2, Wh, att, Ws, bs,
            Wc1, bc1):
    f32 = jnp.float32
    atta = att[:D_H].reshape(D_H, 1)
    attb = att[D_H:].reshape(D_H, 1)
    return pl.pallas_call(
        _tc_pre_body,
        out_shape=[
            jax.ShapeDtypeStruct((N, D_H), f32),
            jax.ShapeDtypeStruct((N, D_H), f32),
            jax.ShapeDtypeStruct((E_H, D_H), f32),
            jax.ShapeDtypeStruct((N, 1), f32),
            jax.ShapeDtypeStruct((E_H, 1), f32),
        ],
    )(x, x_e, W1, b1.reshape(1, D_H), g1.reshape(1, D_H), be1.reshape(1, D_H),
      Wr1, br1.reshape(1, D_H), Wr2, br2.reshape(1, D_H), Wh,
      atta, attb, Ws, bs.reshape(1, D_H), Wc1[D_H:], bc1.reshape(1, D_H))


def _tc_mid1_body(pp_ref, as_ref, bd_ref, q_ref):
    P = pp_ref[...][0, :E_H] + pp_ref[...][1, :E_H]
    asum = as_ref[...][0] + as_ref[...][1] + 1e-16
    bd = bd_ref[...][0] + bd_ref[...][1]
    binv = jnp.where(bd > 0, 1.0 / bd, 0.0)
    scale = binv / (asum * asum)
    q_ref[...] = P * scale[:, None]


def _tc_mid1(pp, asum_p, bd_p):
    return pl.pallas_call(
        _tc_mid1_body,
        out_shape=jax.ShapeDtypeStruct((E_H, D_H), jnp.float32),
    )(pp, asum_p, bd_p)


def _tc_mid2_body(rr_ref, dd_ref, resp_ref, bh_ref, g2_ref, be2_ref, h_ref):
    R = rr_ref[...][0, :N] + rr_ref[...][1, :N]
    dd = dd_ref[...][0] + dd_ref[...][1]
    dinv = jnp.where(dd > 0, 1.0 / dd, 0.0)
    out = R * dinv[:, None] + bh_ref[...]
    h = _lrelu(out)
    m = jnp.mean(h, axis=0, keepdims=True)
    v = jnp.mean((h - m) ** 2, axis=0, keepdims=True)
    h = (h - m) / jnp.sqrt(v + _EPS_BN) * g2_ref[...] + be2_ref[...]
    h_ref[...] = h + resp_ref[...]


def _tc_mid2(rr, dd_p, resp, bh, g2, be2):
    return pl.pallas_call(
        _tc_mid2_body,
        out_shape=jax.ShapeDtypeStruct((N, D_H), jnp.float32),
    )(rr, dd_p, resp, bh.reshape(1, D_H), g2.reshape(1, D_H),
      be2.reshape(1, D_H))


def _tc_post_body(agg_ref, xep_ref, Wc1a_ref, Wc2_ref, bc2_ref, o_ref):
    f32 = jnp.float32
    c = _lrelu(jnp.dot(agg_ref[...], Wc1a_ref[...], preferred_element_type=f32)
               + xep_ref[...])
    o_ref[...] = jnp.dot(c, Wc2_ref[...], preferred_element_type=f32) + bc2_ref[...]


def _tc_post(agg, xep, Wc1, Wc2, bc2):
    return pl.pallas_call(
        _tc_post_body,
        out_shape=jax.ShapeDtypeStruct((E_H, D_OUT), jnp.float32),
    )(agg, xep, Wc1[:D_H], Wc2, bc2.reshape(1, D_OUT))


def _make_conv_pass(scatter_by_col, with_scalars):
    f32 = jnp.float32
    i32 = jnp.int32
    mesh = plsc.VectorSubcoreMesh(core_axis_name="c", subcore_axis_name="s")

    if with_scalars:
        out_type = ([jax.ShapeDtypeStruct((2, _NPAD, D_H), f32)]
                    + [jax.ShapeDtypeStruct((2, N), f32)] * 3)
    else:
        out_type = jax.ShapeDtypeStruct((2, _NPAD, D_H), f32)

    scratch = [
        pltpu.VMEM((_C,), f32),
        pltpu.VMEM((_C,), f32),
        pltpu.VMEM((_C,), i32),
        pltpu.VMEM((_C,), i32),
        pltpu.VMEM((_C,), f32),
        pltpu.VMEM((_C,), f32),
        pltpu.VMEM((_C, D_H), f32),
        pltpu.VMEM_SHARED((_NPAD, D_H), f32),
    ]
    if with_scalars:
        scratch += [pltpu.VMEM_SHARED((N,), f32)] * 3

    def body(*refs):
        if with_scalars:
            (table, row_h, col_h, sn_h, se_h, zr_h, zs_h,
             p_out, asum_o, bd_o, dd_o,
             sn_v, se_v, ridx, cidx, ae_v, ones_v, rows_v, acc,
             asum_s, bd_s, dd_s) = refs
        else:
            (table, row_h, col_h, sn_h, se_h, zr_h,
             p_out,
             sn_v, se_v, ridx, cidx, ae_v, ones_v, rows_v, acc) = refs

        ci = lax.axis_index("c")
        si = lax.axis_index("s")
        wid = si * 2 + ci

        pltpu.sync_copy(zr_h.at[pl.ds(_RPS * si, _RPS)],
                        acc.at[pl.ds(_RPS * si, _RPS)])
        if with_scalars:
            @pl.when(si == 0)
            def _():
                pltpu.sync_copy(zs_h, asum_s)
                pltpu.sync_copy(zs_h, bd_s)
                pltpu.sync_copy(zs_h, dd_s)

        @pl.loop(0, _C, step=16)
        def _(i):
            ones_v[pl.ds(i, 16)] = jnp.full((16,), 1.0, f32)

        plsc.subcore_barrier()

        iot = lax.broadcasted_iota(i32, (16,), 0)
        ebase = wid * _EPW

        @pl.loop(0, _NCHUNK)
        def _(k):
            base = ebase + k * _C
            pltpu.sync_copy(row_h.at[pl.ds(base, _C)], ridx)
            pltpu.sync_copy(col_h.at[pl.ds(base, _C)], cidx)
            gidx = ridx if scatter_by_col else cidx
            sidx = cidx if scatter_by_col else ridx
            pltpu.sync_copy(table.at[gidx], rows_v)
            pltpu.sync_copy(sn_h.at[ridx], sn_v)
            pltpu.sync_copy(se_h.at[cidx], se_v)

            @pl.loop(0, _C, step=16)
            def _(g):
                s = sn_v[pl.ds(g, 16)] + se_v[pl.ds(g, 16)]
                ae_v[pl.ds(g, 16)] = jnp.exp(jnp.maximum(s, 0.2 * s))

            @pl.loop(0, _C)
            def _(e):
                ev = jnp.full((16,), e, i32)
                wv = plsc.load_gather(ae_v, [ev])
                for j in range(8):
                    fj = iot + (16 * j)
                    r = plsc.load_gather(rows_v, [ev, fj])
                    plsc.store_scatter(rows_v, [ev, fj], r * wv)

            pltpu.sync_copy(rows_v, acc.at[sidx], add=True)
            if with_scalars:
                pltpu.sync_copy(ae_v, asum_s.at[cidx], add=True)
                pltpu.sync_copy(ones_v, bd_s.at[cidx], add=True)
                pltpu.sync_copy(ones_v, dd_s.at[ridx], add=True)

        plsc.subcore_barrier()
        pltpu.sync_copy(acc.at[pl.ds(_RPS * si, _RPS)],
                        p_out.at[ci, pl.ds(_RPS * si, _RPS)])
        if with_scalars:
            @pl.when(si == 0)
            def _():
                pltpu.sync_copy(asum_s, asum_o.at[ci])
                pltpu.sync_copy(bd_s, bd_o.at[ci])
                pltpu.sync_copy(dd_s, dd_o.at[ci])

    cp = pltpu.CompilerParams()
    if "needs_layout_passes" in pltpu.CompilerParams.__dataclass_fields__:
        cp = dataclasses.replace(cp, needs_layout_passes=False)
    return pl.kernel(body, out_type=out_type, mesh=mesh,
                     scratch_types=scratch, compiler_params=cp)


_conv_pass_b = _make_conv_pass(scatter_by_col=True, with_scalars=True)
_conv_pass_c = _make_conv_pass(scatter_by_col=False, with_scalars=False)


_SEG = 320
_CS = 2000
_NSC = NNZ // _CS
_PB = 256
_PCAP = _PB + 16


def _make_segmin():
    f32 = jnp.float32
    i32 = jnp.int32
    mesh = plsc.VectorSubcoreMesh(core_axis_name="c", subcore_axis_name="s")

    out_type = jax.ShapeDtypeStruct((N, D_H), f32)
    scratch = [
        pltpu.VMEM((_SEG, D_H), f32),
        pltpu.VMEM((_CS,), i32),
        pltpu.VMEM((_CS,), i32),
        pltpu.VMEM((_PCAP,), i32),
        pltpu.VMEM((_PCAP,), i32),
        pltpu.VMEM((_PCAP, D_H), f32),
    ]

    def body(h_hbm, row_h, col_h, inf_h, agg_out,
             agg_loc, rbuf, cbuf, prow, pcol, rows_p):
        ci = lax.axis_index("c")
        si = lax.axis_index("s")
        wid = si * 2 + ci
        lo = wid * _SEG

        pltpu.sync_copy(inf_h, agg_loc)

        @pl.loop(0, _PCAP, step=16)
        def _(i):
            prow[pl.ds(i, 16)] = jnp.zeros((16,), i32)
            pcol[pl.ds(i, 16)] = jnp.zeros((16,), i32)

        iot = lax.broadcasted_iota(i32, (16,), 0)
        fjs = [iot + 16 * j for j in range(8)]
        lov = jnp.full((16,), lo, i32)

        def drain(cnt):
            pltpu.sync_copy(h_hbm.at[prow], rows_p)
            cntv = jnp.full((16,), cnt, i32)

            @pl.loop(0, _PCAP)
            def _(e):
                ev = jnp.full((16,), e, i32)
                valid = ev < cntv
                cl = plsc.load_gather(pcol, [ev])
                for j in range(8):
                    hr = plsc.load_gather(rows_p, [ev, fjs[j]])
                    old = plsc.load_gather(agg_loc, [cl, fjs[j]])
                    plsc.store_scatter(agg_loc, [cl, fjs[j]],
                                       jnp.minimum(old, hr), mask=valid)

        def vec_body(g, cnt):
            g16 = g * 16
            rv = rbuf[pl.ds(g16, 16)]
            cv = cbuf[pl.ds(g16, 16)]
            m = (cv >= lov) & (cv < lov + _SEG)
            mi = m.astype(i32)
            pos = jnp.full((16,), cnt, i32) + plsc.cumsum(mi) - 1
            plsc.store_scatter(pcol, [pos], cv - lov, mask=m)
            plsc.store_scatter(prow, [pos], rv, mask=m)
            cnt = cnt + jnp.sum(mi)
            did = (cnt >= _PB).astype(i32)

            @pl.when(did == 1)
            def _():
                drain(_PB)
                left = jnp.full((16,), cnt - _PB, i32)
                lm = iot < left
                lc = plsc.load_gather(pcol, [iot + _PB])
                lr = plsc.load_gather(prow, [iot + _PB])
                plsc.store_scatter(pcol, [iot], lc, mask=lm)
                plsc.store_scatter(prow, [iot], lr, mask=lm)

            return cnt - did * _PB

        def chunk_body(k, cnt):
            base = k * _CS
            pltpu.sync_copy(row_h.at[pl.ds(base, _CS)], rbuf)
            pltpu.sync_copy(col_h.at[pl.ds(base, _CS)], cbuf)
            return lax.fori_loop(0, _CS // 16, vec_body, cnt)

        cnt = lax.fori_loop(0, _NSC, chunk_body, 0)
        drain(cnt)

        @pl.when(lo + _SEG <= N)
        def _():
            pltpu.sync_copy(agg_loc, agg_out.at[pl.ds(lo, _SEG)])

        @pl.when(lo + _SEG > N)
        def _():
            pltpu.sync_copy(agg_loc.at[pl.ds(0, N - _SEG * 31)],
                            agg_out.at[pl.ds(lo, N - _SEG * 31)])

    cp = pltpu.CompilerParams()
    if "needs_layout_passes" in pltpu.CompilerParams.__dataclass_fields__:
        cp = dataclasses.replace(cp, needs_layout_passes=False)
    return pl.kernel(body, out_type=out_type, mesh=mesh,
                     scratch_types=scratch, compiler_params=cp)


_sc_segmin = _make_segmin()


def kernel(x, x_struct, x_e, edge_index, W1, b1, g1, be1, Wr1, br1, Wr2, br2,
           Wh, att, bh, g2, be2, Ws, bs, Wc1, bc1, Wc2, bc2):
    f32 = jnp.float32
    row = edge_index[0]
    col = edge_index[1]

    xt, resp, xep, sn, se = _tc_pre(x, x_e, W1, b1, g1, be1, Wr1, br1, Wr2,
                                    br2, Wh, att, Ws, bs, Wc1, bc1)
    sn1 = sn.reshape(N)
    se1 = se.reshape(E_H)

    zrows = jnp.zeros((_NPAD, D_H), f32)
    zscal = jnp.zeros((N,), f32)

    pp, asum_p, bd_p, dd_p = _conv_pass_b(xt, row, col, sn1, se1, zrows, zscal)
    q = _tc_mid1(pp, asum_p, bd_p)
    rr = _conv_pass_c(q, row, col, sn1, se1, zrows)
    h = _tc_mid2(rr, dd_p, resp, bh, g2, be2)

    inf_h = jnp.full((_SEG, D_H), jnp.inf, f32)
    agg = _sc_segmin(h, row, col, inf_h)

    return _tc_post(agg, xep, Wc1, Wc2, bc2)


# speedup vs baseline: 1.2509x; 1.2509x over previous
import dataclasses
import functools

import jax
import jax.numpy as jnp
from jax import lax
from jax.experimental import pallas as pl
from jax.experimental.pallas import tpu as pltpu
from jax.experimental.pallas import tpu_sc as plsc

N = 10000
E_H = 10000
NNZ = 320000
D_IN = 128
D_H = 128
D_OUT = 64
_EPS_BN = 1e-5

_NW = 32
_EPW = NNZ // _NW
_C = 200
_NCHUNK = _EPW // _C
_NPAD = 10240
_RPS = _NPAD // 16


def _lrelu(x, slope=0.01):
    return jnp.where(x >= 0, x, slope * x)


def _tc_pre_body(x_ref, xe_in_ref, W1_ref, b1_ref, g1_ref, be1_ref,
                 Wr1_ref, br1_ref, Wr2_ref, br2_ref, Wh_ref,
                 atta_ref, attb_ref, Ws_ref, bs_ref, Wc1b_ref, bc1_ref,
                 xt_ref, rp_ref, xep_ref, sn_ref, se_ref):
    f32 = jnp.float32
    xb = _lrelu(jnp.dot(x_ref[...], W1_ref[...], preferred_element_type=f32)
                + b1_ref[...])
    m = jnp.mean(xb, axis=0, keepdims=True)
    v = jnp.mean((xb - m) ** 2, axis=0, keepdims=True)
    xb = (xb - m) / jnp.sqrt(v + _EPS_BN) * g1_ref[...] + be1_ref[...]

    xe1 = _lrelu(jnp.dot(xe_in_ref[...], Wr1_ref[...], preferred_element_type=f32)
                 + br1_ref[...])
    xe = jnp.dot(xe1, Wr2_ref[...], preferred_element_type=f32) + br2_ref[...]

    xt = jnp.dot(xb, Wh_ref[...], preferred_element_type=f32)
    he = jnp.dot(xe, Wh_ref[...], preferred_element_type=f32)

    xt_ref[...] = xt
    sn_ref[...] = jnp.dot(xt, atta_ref[...], preferred_element_type=f32)
    se_ref[...] = jnp.dot(he, attb_ref[...], preferred_element_type=f32)
    rp_ref[...] = jnp.dot(xb, Ws_ref[...], preferred_element_type=f32) + bs_ref[...]
    xep_ref[...] = (jnp.dot(xe, Wc1b_ref[...], preferred_element_type=f32)
                    + bc1_ref[...])


def _tc_pre(x, x_e, W1, b1, g1, be1, Wr1, br1, Wr2, br2, Wh, att, Ws, bs,
            Wc1, bc1):
    f32 = jnp.float32
    atta = att[:D_H].reshape(D_H, 1)
    attb = att[D_H:].reshape(D_H, 1)
    return pl.pallas_call(
        _tc_pre_body,
        out_shape=[
            jax.ShapeDtypeStruct((N, D_H), f32),
            jax.ShapeDtypeStruct((N, D_H), f32),
            jax.ShapeDtypeStruct((E_H, D_H), f32),
            jax.ShapeDtypeStruct((N, 1), f32),
            jax.ShapeDtypeStruct((E_H, 1), f32),
        ],
    )(x, x_e, W1, b1.reshape(1, D_H), g1.reshape(1, D_H), be1.reshape(1, D_H),
      Wr1, br1.reshape(1, D_H), Wr2, br2.reshape(1, D_H), Wh,
      atta, attb, Ws, bs.reshape(1, D_H), Wc1[D_H:], bc1.reshape(1, D_H))


def _tc_mid1_body(pp_ref, as_ref, bd_ref, q_ref):
    P = pp_ref[...][0, :E_H] + pp_ref[...][1, :E_H]
    asum = as_ref[...][0] + as_ref[...][1] + 1e-16
    bd = bd_ref[...][0] + bd_ref[...][1]
    binv = jnp.where(bd > 0, 1.0 / bd, 0.0)
    scale = binv / (asum * asum)
    q_ref[...] = P * scale[:, None]


def _tc_mid1(pp, asum_p, bd_p):
    return pl.pallas_call(
        _tc_mid1_body,
        out_shape=jax.ShapeDtypeStruct((E_H, D_H), jnp.float32),
    )(pp, asum_p, bd_p)


def _tc_mid2_body(rr_ref, dd_ref, resp_ref, bh_ref, g2_ref, be2_ref, h_ref):
    R = rr_ref[...][0, :N] + rr_ref[...][1, :N]
    dd = dd_ref[...][0] + dd_ref[...][1]
    dinv = jnp.where(dd > 0, 1.0 / dd, 0.0)
    out = R * dinv[:, None] + bh_ref[...]
    h = _lrelu(out)
    m = jnp.mean(h, axis=0, keepdims=True)
    v = jnp.mean((h - m) ** 2, axis=0, keepdims=True)
    h = (h - m) / jnp.sqrt(v + _EPS_BN) * g2_ref[...] + be2_ref[...]
    h_ref[...] = h + resp_ref[...]


def _tc_mid2(rr, dd_p, resp, bh, g2, be2):
    return pl.pallas_call(
        _tc_mid2_body,
        out_shape=jax.ShapeDtypeStruct((N, D_H), jnp.float32),
    )(rr, dd_p, resp, bh.reshape(1, D_H), g2.reshape(1, D_H),
      be2.reshape(1, D_H))


def _tc_post_body(agg_ref, xep_ref, Wc1a_ref, Wc2_ref, bc2_ref, o_ref):
    f32 = jnp.float32
    c = _lrelu(jnp.dot(agg_ref[...], Wc1a_ref[...], preferred_element_type=f32)
               + xep_ref[...])
    o_ref[...] = jnp.dot(c, Wc2_ref[...], preferred_element_type=f32) + bc2_ref[...]


def _tc_post(agg, xep, Wc1, Wc2, bc2):
    return pl.pallas_call(
        _tc_post_body,
        out_shape=jax.ShapeDtypeStruct((E_H, D_OUT), jnp.float32),
    )(agg, xep, Wc1[:D_H], Wc2, bc2.reshape(1, D_OUT))


def _make_conv_pass(scatter_by_col, with_scalars):
    f32 = jnp.float32
    i32 = jnp.int32
    mesh = plsc.VectorSubcoreMesh(core_axis_name="c", subcore_axis_name="s")

    if with_scalars:
        out_type = ([jax.ShapeDtypeStruct((2, _NPAD, D_H), f32)]
                    + [jax.ShapeDtypeStruct((2, N), f32)] * 3)
    else:
        out_type = jax.ShapeDtypeStruct((2, _NPAD, D_H), f32)

    scratch = [
        pltpu.VMEM((_C,), f32),
        pltpu.VMEM((_C,), f32),
        pltpu.VMEM((_C,), i32),
        pltpu.VMEM((_C,), i32),
        pltpu.VMEM((_C,), f32),
        pltpu.VMEM((_C,), f32),
        pltpu.VMEM((_C, D_H), f32),
        pltpu.VMEM_SHARED((_NPAD, D_H), f32),
    ]
    if with_scalars:
        scratch += [pltpu.VMEM_SHARED((N,), f32)] * 3

    def body(*refs):
        if with_scalars:
            (table, row_h, col_h, sn_h, se_h, zr_h, zs_h,
             p_out, asum_o, bd_o, dd_o,
             sn_v, se_v, ridx, cidx, ae_v, ones_v, rows_v, acc,
             asum_s, bd_s, dd_s) = refs
        else:
            (table, row_h, col_h, sn_h, se_h, zr_h,
             p_out,
             sn_v, se_v, ridx, cidx, ae_v, ones_v, rows_v, acc) = refs

        ci = lax.axis_index("c")
        si = lax.axis_index("s")
        wid = si * 2 + ci

        pltpu.sync_copy(zr_h.at[pl.ds(_RPS * si, _RPS)],
                        acc.at[pl.ds(_RPS * si, _RPS)])
        if with_scalars:
            @pl.when(si == 0)
            def _():
                pltpu.sync_copy(zs_h, asum_s)
                pltpu.sync_copy(zs_h, bd_s)
                pltpu.sync_copy(zs_h, dd_s)

        @pl.loop(0, _C, step=16)
        def _(i):
            ones_v[pl.ds(i, 16)] = jnp.full((16,), 1.0, f32)

        plsc.subcore_barrier()

        iot = lax.broadcasted_iota(i32, (16,), 0)
        ebase = wid * _EPW

        @pl.loop(0, _NCHUNK)
        def _(k):
            base = ebase + k * _C
            pltpu.sync_copy(row_h.at[pl.ds(base, _C)], ridx)
            pltpu.sync_copy(col_h.at[pl.ds(base, _C)], cidx)
            gidx = ridx if scatter_by_col else cidx
            sidx = cidx if scatter_by_col else ridx
            pltpu.sync_copy(table.at[gidx], rows_v)
            pltpu.sync_copy(sn_h.at[ridx], sn_v)
            pltpu.sync_copy(se_h.at[cidx], se_v)

            @pl.loop(0, _C, step=16)
            def _(g):
                s = sn_v[pl.ds(g, 16)] + se_v[pl.ds(g, 16)]
                ae_v[pl.ds(g, 16)] = jnp.exp(jnp.maximum(s, 0.2 * s))

            @pl.loop(0, _C)
            def _(e):
                ev = jnp.full((16,), e, i32)
                wv = plsc.load_gather(ae_v, [ev])
                for j in range(8):
                    sl = pl.ds(16 * j, 16)
                    rows_v[e, sl] = rows_v[e, sl] * wv

            pltpu.sync_copy(rows_v, acc.at[sidx], add=True)
            if with_scalars:
                pltpu.sync_copy(ae_v, asum_s.at[cidx], add=True)
                pltpu.sync_copy(ones_v, bd_s.at[cidx], add=True)
                pltpu.sync_copy(ones_v, dd_s.at[ridx], add=True)

        plsc.subcore_barrier()
        pltpu.sync_copy(acc.at[pl.ds(_RPS * si, _RPS)],
                        p_out.at[ci, pl.ds(_RPS * si, _RPS)])
        if with_scalars:
            @pl.when(si == 0)
            def _():
                pltpu.sync_copy(asum_s, asum_o.at[ci])
                pltpu.sync_copy(bd_s, bd_o.at[ci])
                pltpu.sync_copy(dd_s, dd_o.at[ci])

    cp = pltpu.CompilerParams()
    if "needs_layout_passes" in pltpu.CompilerParams.__dataclass_fields__:
        cp = dataclasses.replace(cp, needs_layout_passes=False)
    return pl.kernel(body, out_type=out_type, mesh=mesh,
                     scratch_types=scratch, compiler_params=cp)


_conv_pass_b = _make_conv_pass(scatter_by_col=True, with_scalars=True)
_conv_pass_c = _make_conv_pass(scatter_by_col=False, with_scalars=False)


_SEG = 320
_CS = 2000
_NSC = NNZ // _CS
_PB = 256
_PCAP = _PB + 16


def _make_segmin():
    f32 = jnp.float32
    i32 = jnp.int32
    mesh = plsc.VectorSubcoreMesh(core_axis_name="c", subcore_axis_name="s")

    out_type = jax.ShapeDtypeStruct((N, D_H), f32)
    scratch = [
        pltpu.VMEM((_SEG, D_H), f32),
        pltpu.VMEM((_CS,), i32),
        pltpu.VMEM((_CS,), i32),
        pltpu.VMEM((_PCAP,), i32),
        pltpu.VMEM((_PCAP,), i32),
        pltpu.VMEM((_PCAP, D_H), f32),
    ]

    def body(h_hbm, row_h, col_h, inf_h, agg_out,
             agg_loc, rbuf, cbuf, prow, pcol, rows_p):
        ci = lax.axis_index("c")
        si = lax.axis_index("s")
        wid = si * 2 + ci
        lo = wid * _SEG

        pltpu.sync_copy(inf_h, agg_loc)

        @pl.loop(0, _PCAP, step=16)
        def _(i):
            prow[pl.ds(i, 16)] = jnp.zeros((16,), i32)
            pcol[pl.ds(i, 16)] = jnp.zeros((16,), i32)

        iot = lax.broadcasted_iota(i32, (16,), 0)
        fjs = [iot + 16 * j for j in range(8)]
        lov = jnp.full((16,), lo, i32)

        def drain(cnt):
            pltpu.sync_copy(h_hbm.at[prow], rows_p)
            cntv = jnp.full((16,), cnt, i32)

            @pl.loop(0, _PCAP)
            def _(e):
                ev = jnp.full((16,), e, i32)
                valid = ev < cntv
                cl = plsc.load_gather(pcol, [ev])
                for j in range(8):
                    hr = rows_p[e, pl.ds(16 * j, 16)]
                    old = plsc.load_gather(agg_loc, [cl, fjs[j]])
                    plsc.store_scatter(agg_loc, [cl, fjs[j]],
                                       jnp.minimum(old, hr), mask=valid)

        def vec_body(g, cnt):
            g16 = g * 16
            rv = rbuf[pl.ds(g16, 16)]
            cv = cbuf[pl.ds(g16, 16)]
            m = (cv >= lov) & (cv < lov + _SEG)
            mi = m.astype(i32)
            tot = jnp.sum(mi)

            @pl.when(tot != 0)
            def _():
                pos = jnp.full((16,), cnt, i32) + plsc.cumsum(mi) - 1
                plsc.store_scatter(pcol, [pos], cv - lov, mask=m)
                plsc.store_scatter(prow, [pos], rv, mask=m)

            cnt = cnt + tot
            did = (cnt >= _PB).astype(i32)

            @pl.when(did == 1)
            def _():
                drain(_PB)
                left = jnp.full((16,), cnt - _PB, i32)
                lm = iot < left
                lc = plsc.load_gather(pcol, [iot + _PB])
                lr = plsc.load_gather(prow, [iot + _PB])
                plsc.store_scatter(pcol, [iot], lc, mask=lm)
                plsc.store_scatter(prow, [iot], lr, mask=lm)

            return cnt - did * _PB

        def chunk_body(k, cnt):
            base = k * _CS
            pltpu.sync_copy(row_h.at[pl.ds(base, _CS)], rbuf)
            pltpu.sync_copy(col_h.at[pl.ds(base, _CS)], cbuf)
            return lax.fori_loop(0, _CS // 16, vec_body, cnt)

        cnt = lax.fori_loop(0, _NSC, chunk_body, 0)
        drain(cnt)

        @pl.when(lo + _SEG <= N)
        def _():
            pltpu.sync_copy(agg_loc, agg_out.at[pl.ds(lo, _SEG)])

        @pl.when(lo + _SEG > N)
        def _():
            pltpu.sync_copy(agg_loc.at[pl.ds(0, N - _SEG * 31)],
                            agg_out.at[pl.ds(lo, N - _SEG * 31)])

    cp = pltpu.CompilerParams()
    if "needs_layout_passes" in pltpu.CompilerParams.__dataclass_fields__:
        cp = dataclasses.replace(cp, needs_layout_passes=False)
    return pl.kernel(body, out_type=out_type, mesh=mesh,
                     scratch_types=scratch, compiler_params=cp)


_sc_segmin = _make_segmin()


def kernel(x, x_struct, x_e, edge_index, W1, b1, g1, be1, Wr1, br1, Wr2, br2,
           Wh, att, bh, g2, be2, Ws, bs, Wc1, bc1, Wc2, bc2):
    f32 = jnp.float32
    row = edge_index[0]
    col = edge_index[1]

    xt, resp, xep, sn, se = _tc_pre(x, x_e, W1, b1, g1, be1, Wr1, br1, Wr2,
                                    br2, Wh, att, Ws, bs, Wc1, bc1)
    sn1 = sn.reshape(N)
    se1 = se.reshape(E_H)

    zrows = jnp.zeros((_NPAD, D_H), f32)
    zscal = jnp.zeros((N,), f32)

    pp, asum_p, bd_p, dd_p = _conv_pass_b(xt, row, col, sn1, se1, zrows, zscal)
    q = _tc_mid1(pp, asum_p, bd_p)
    rr = _conv_pass_c(q, row, col, sn1, se1, zrows)
    h = _tc_mid2(rr, dd_p, resp, bh, g2, be2)

    inf_h = jnp.full((_SEG, D_H), jnp.inf, f32)
    agg = _sc_segmin(h, row, col, inf_h)

    return _tc_post(agg, xep, Wc1, Wc2, bc2)
